# Initial kernel scaffold; baseline (speedup 1.0000x reference)
import jax
import jax.numpy as jnp
from jax.experimental import pallas as pl


def kernel(x, edge_index, W1, att_s1, att_d1, b1, W2, att_s2, att_d2, b2, Wp, bp):
    raise NotImplementedError("write your pallas kernel here")


# speedup vs baseline: 33.4251x; 33.4251x over previous
import functools

import jax
import jax.numpy as jnp
from jax import lax
from jax.experimental import pallas as pl
from jax.experimental.pallas import tpu as pltpu
from jax.experimental.pallas import tpu_sc as plsc

_N = 10000
_E = 320000
_D = 128
_DH = 64
_HW = 80
_CH = 80
_NW = 32
_EPW = _E // _NW
_EPT = _E // 16
_NCH = _EPT // _CH
_RPT = 640
_R = 2000


def _leaky(x):
    return jnp.where(x >= 0, x, 0.2 * x)


def _feature_tail(h, as_ref, ad_ref, h_ref, s_ref, d_ref):
    s_ref[...] = jnp.sum(h * as_ref[...], axis=1, keepdims=True)
    d_ref[...] = jnp.sum(h * ad_ref[...], axis=1, keepdims=True)
    ones = jnp.ones((_R, 1), jnp.float32)
    pad = jnp.zeros((_R, _HW - _DH - 1), jnp.float32)
    h_ref[0, ...] = jnp.concatenate([h[:, :_DH], ones, pad], axis=1)
    h_ref[1, ...] = jnp.concatenate([h[:, _DH:], ones, pad], axis=1)


def _prev_h(h_ref):
    return jnp.concatenate([h_ref[0, :, :_DH], h_ref[1, :, :_DH]], axis=1)


def _tc_pre_body(x_ref, w_ref, as_ref, ad_ref, h_ref, s_ref, d_ref):
    h = jnp.dot(x_ref[...], w_ref[...], preferred_element_type=jnp.float32)
    _feature_tail(h, as_ref, ad_ref, h_ref, s_ref, d_ref)


_ROWB = pl.BlockSpec((_R, 1), lambda i: (i, 0))
_H3B = pl.BlockSpec((2, _R, _HW), lambda i: (0, i, 0))


def _tc_pre(x, w, att_s, att_d):
    return pl.pallas_call(
        _tc_pre_body,
        grid=(_N // _R,),
        in_specs=[
            pl.BlockSpec((_R, _D), lambda i: (i, 0)),
            pl.BlockSpec((_D, _D), lambda i: (0, 0)),
            pl.BlockSpec((1, _D), lambda i: (0, 0)),
            pl.BlockSpec((1, _D), lambda i: (0, 0)),
        ],
        out_specs=(_H3B, _ROWB, _ROWB),
        out_shape=(
            jax.ShapeDtypeStruct((2, _N, _HW), jnp.float32),
            jax.ShapeDtypeStruct((_N, 1), jnp.float32),
            jax.ShapeDtypeStruct((_N, 1), jnp.float32),
        ),
    )(x, w, att_s, att_d)


def _normalized(acc_ref, h_ref, s_ref, d_ref, b_ref):
    den0 = jnp.exp(_leaky(s_ref[...] + d_ref[...]))
    num = jnp.concatenate([acc_ref[0, :, :_DH], acc_ref[1, :, :_DH]], axis=1)
    num = num + den0 * _prev_h(h_ref)
    den = acc_ref[0, :, _DH:_DH + 1] + den0
    return num / den + b_ref[...]


def _tc_mid_body(acc_ref, h_ref, s_ref, d_ref, b_ref, w_ref, as_ref, ad_ref,
                 h2_ref, s2_ref, d2_ref):
    h1 = jnp.maximum(_normalized(acc_ref, h_ref, s_ref, d_ref, b_ref), 0.0)
    h2 = jnp.dot(h1, w_ref[...], preferred_element_type=jnp.float32)
    _feature_tail(h2, as_ref, ad_ref, h2_ref, s2_ref, d2_ref)


def _tc_mid(acc, h, s, d, b, w, att_s, att_d):
    return pl.pallas_call(
        _tc_mid_body,
        grid=(_N // _R,),
        in_specs=[
            _H3B,
            _H3B,
            _ROWB,
            _ROWB,
            pl.BlockSpec((1, _D), lambda i: (0, 0)),
            pl.BlockSpec((_D, _D), lambda i: (0, 0)),
            pl.BlockSpec((1, _D), lambda i: (0, 0)),
            pl.BlockSpec((1, _D), lambda i: (0, 0)),
        ],
        out_specs=(_H3B, _ROWB, _ROWB),
        out_shape=(
            jax.ShapeDtypeStruct((2, _N, _HW), jnp.float32),
            jax.ShapeDtypeStruct((_N, 1), jnp.float32),
            jax.ShapeDtypeStruct((_N, 1), jnp.float32),
        ),
    )(acc, h, s, d, b, w, att_s, att_d)


def _tc_post_body(acc_ref, h_ref, s_ref, d_ref, b_ref, wp_ref, bp_ref,
                  ps_ref, pd_ref):
    hout = _normalized(acc_ref, h_ref, s_ref, d_ref, b_ref)
    ps_ref[...] = jnp.dot(hout, wp_ref[:_D, :],
                          preferred_element_type=jnp.float32) + bp_ref[...]
    pd_ref[...] = jnp.dot(hout, wp_ref[_D:, :],
                          preferred_element_type=jnp.float32)


def _tc_post(acc, h, s, d, b, wp, bp):
    return pl.pallas_call(
        _tc_post_body,
        grid=(_N // _R,),
        in_specs=[
            _H3B,
            _H3B,
            _ROWB,
            _ROWB,
            pl.BlockSpec((1, _D), lambda i: (0, 0)),
            pl.BlockSpec((2 * _D, 1), lambda i: (0, 0)),
            pl.BlockSpec((1, 1), lambda i: (0, 0)),
        ],
        out_specs=(_ROWB, _ROWB),
        out_shape=(
            jax.ShapeDtypeStruct((_N, 1), jnp.float32),
            jax.ShapeDtypeStruct((_N, 1), jnp.float32),
        ),
    )(acc, h, s, d, b, wp, bp)


_MESH = plsc.VectorSubcoreMesh(core_axis_name="c", subcore_axis_name="s")
_SC_PARAMS = pltpu.CompilerParams(needs_layout_passes=False,
                                  use_tc_tiling_on_sc=False)


@functools.partial(
    pl.kernel,
    out_type=jax.ShapeDtypeStruct((2, _N, _HW), jnp.float32),
    mesh=_MESH,
    compiler_params=_SC_PARAMS,
    scratch_types=[
        pltpu.VMEM((_N,), jnp.float32),
        pltpu.VMEM((_N,), jnp.float32),
        pltpu.VMEM((_EPT,), jnp.int32),
        pltpu.VMEM((_EPT,), jnp.int32),
        pltpu.VMEM((_CH,), jnp.int32),
        pltpu.VMEM((_CH,), jnp.int32),
        pltpu.VMEM((_CH,), jnp.float32),
        pltpu.VMEM((_CH,), jnp.float32),
        pltpu.VMEM((_CH, _HW), jnp.float32),
        pltpu.VMEM((_CH, _HW), jnp.float32),
        pltpu.VMEM_SHARED((_N, _HW), jnp.float32),
        pltpu.SemaphoreType.DMA,
        pltpu.SemaphoreType.DMA,
        pltpu.SemaphoreType.DMA,
        pltpu.SemaphoreType.DMA,
    ],
)
def _sc_agg(src_hbm, dst_hbm, s_hbm, d_hbm, h_hbm, z_hbm, acc_hbm,
            s_v, d_v, src_v, src2_v, dsta, dstb, exa, exb, rowsa, rowsb,
            num_sh, sema, semb, semia, semib):
    cid = lax.axis_index("c")
    sid = lax.axis_index("s")
    base = sid * _EPT

    pltpu.sync_copy(s_hbm, s_v)
    pltpu.sync_copy(d_hbm, d_v)
    pltpu.sync_copy(src_hbm.at[pl.ds(base, _EPT)], src_v)
    bias = jnp.broadcast_to(cid * _N, (16,)).astype(jnp.int32)

    def biasbody(i, carry):
        src2_v[pl.ds(i * 16, 16)] = src_v[pl.ds(i * 16, 16)] + bias
        return carry

    lax.fori_loop(0, _EPT // 16, biasbody, 0)

    @pl.when(sid < 15)
    def _():
        pltpu.sync_copy(z_hbm.at[pl.ds(sid * _RPT, _RPT)],
                        num_sh.at[pl.ds(sid * _RPT, _RPT)])

    @pl.when(sid == 15)
    def _():
        pltpu.sync_copy(z_hbm.at[pl.ds(15 * _RPT, _N - 15 * _RPT)],
                        num_sh.at[pl.ds(15 * _RPT, _N - 15 * _RPT)])

    plsc.subcore_barrier()

    def issue_gather(ci, rows, dstb_, sem, semi):
        idx = src2_v.at[pl.ds(ci * _CH, _CH)]
        pltpu.async_copy(h_hbm.at[idx], rows, sem)
        pltpu.async_copy(dst_hbm.at[pl.ds(base + ci * _CH, _CH)], dstb_, semi)

    def wait_gather(ci, rows, dstb_, sem, semi):
        idx = src2_v.at[pl.ds(ci * _CH, _CH)]
        pltpu.make_async_copy(h_hbm.at[idx], rows, sem).wait()
        pltpu.make_async_copy(
            dst_hbm.at[pl.ds(base + ci * _CH, _CH)], dstb_, semi).wait()

    def compute_ex(ci, ex, dstb_):
        for j in range(_CH // 16):
            si = src_v[pl.ds(ci * _CH + j * 16, 16)]
            di = dstb_[pl.ds(j * 16, 16)]
            a = plsc.load_gather(s_v, [si]) + plsc.load_gather(d_v, [di])
            ex[pl.ds(j * 16, 16)] = jnp.exp(_leaky(a))

    def scale_rows(rows, ex):
        def body(r, carry):
            er = plsc.load_gather(ex, [jnp.full((16,), r, jnp.int32)])
            for cb in range(_HW // 16):
                rows[r, pl.ds(cb * 16, 16)] = rows[r, pl.ds(cb * 16, 16)] * er
            return carry
        lax.fori_loop(0, _CH, body, 0)

    def scatter(rows, dstb_):
        pltpu.sync_copy(rows, num_sh.at[dstb_], add=True)

    def process(ci, ex, rows, dstb_):
        compute_ex(ci, ex, dstb_)
        scale_rows(rows, ex)
        scatter(rows, dstb_)

    issue_gather(0, rowsa, dsta, sema, semia)

    def pair(o, carry):
        e = o * 2
        wait_gather(e, rowsa, dsta, sema, semia)
        issue_gather(e + 1, rowsb, dstb, semb, semib)
        process(e, exa, rowsa, dsta)
        wait_gather(e + 1, rowsb, dstb, semb, semib)

        @pl.when(e + 2 < _NCH)
        def _():
            issue_gather(e + 2, rowsa, dsta, sema, semia)

        process(e + 1, exb, rowsb, dstb)
        return carry

    lax.fori_loop(0, _NCH // 2, pair, 0)

    plsc.subcore_barrier()

    @pl.when(sid < 15)
    def _():
        pltpu.sync_copy(num_sh.at[pl.ds(sid * _RPT, _RPT)],
                        acc_hbm.at[cid, pl.ds(sid * _RPT, _RPT)])

    @pl.when(sid == 15)
    def _():
        pltpu.sync_copy(num_sh.at[pl.ds(15 * _RPT, _N - 15 * _RPT)],
                        acc_hbm.at[cid, pl.ds(15 * _RPT, _N - 15 * _RPT)])


@functools.partial(
    pl.kernel,
    out_type=jax.ShapeDtypeStruct((_E,), jnp.float32),
    mesh=_MESH,
    compiler_params=_SC_PARAMS,
    scratch_types=[
        pltpu.VMEM((_N,), jnp.float32),
        pltpu.VMEM((_N,), jnp.float32),
        pltpu.VMEM((_EPW,), jnp.int32),
        pltpu.VMEM((_EPW,), jnp.int32),
        pltpu.VMEM((_EPW,), jnp.float32),
    ],
)
def _sc_score(src_hbm, dst_hbm, ps_hbm, pd_hbm, out_hbm,
              ps_v, pd_v, src_v, dst_v, out_v):
    cid = lax.axis_index("c")
    sid = lax.axis_index("s")
    wid = cid * 16 + sid

    pltpu.sync_copy(ps_hbm, ps_v)
    pltpu.sync_copy(pd_hbm, pd_v)
    pltpu.sync_copy(src_hbm.at[pl.ds(wid * _EPW, _EPW)], src_v)
    pltpu.sync_copy(dst_hbm.at[pl.ds(wid * _EPW, _EPW)], dst_v)

    def body(i, carry):
        si = src_v[pl.ds(i * 16, 16)]
        di = dst_v[pl.ds(i * 16, 16)]
        out_v[pl.ds(i * 16, 16)] = (plsc.load_gather(ps_v, [si])
                                    + plsc.load_gather(pd_v, [di]))
        return carry

    lax.fori_loop(0, _EPW // 16, body, 0)
    pltpu.sync_copy(out_v, out_hbm.at[pl.ds(wid * _EPW, _EPW)])


def kernel(x, edge_index, W1, att_s1, att_d1, b1, W2, att_s2, att_d2, b2,
           Wp, bp):
    src = edge_index[0]
    dst = edge_index[1]
    as1 = att_s1.reshape(1, _D)
    ad1 = att_d1.reshape(1, _D)
    as2 = att_s2.reshape(1, _D)
    ad2 = att_d2.reshape(1, _D)
    b1r = b1.reshape(1, _D)
    b2r = b2.reshape(1, _D)
    bpr = bp.reshape(1, 1)
    zeros = jnp.zeros((_N, _HW), jnp.float32)

    h1, s1, d1 = _tc_pre(x, W1, as1, ad1)
    acc1 = _sc_agg(src, dst, s1.reshape(_N), d1.reshape(_N),
                   h1.reshape(2 * _N, _HW), zeros)
    h2, s2, d2 = _tc_mid(acc1, h1, s1, d1, b1r, W2, as2, ad2)
    acc2 = _sc_agg(src, dst, s2.reshape(_N), d2.reshape(_N),
                   h2.reshape(2 * _N, _HW), zeros)
    ps, pd = _tc_post(acc2, h2, s2, d2, b2r, Wp, bpr)
    scores = _sc_score(src, dst, ps.reshape(_N), pd.reshape(_N))
    return scores.reshape(_E, 1)


# speedup vs baseline: 35.6671x; 1.0671x over previous
import functools

import jax
import jax.numpy as jnp
from jax import lax
from jax.experimental import pallas as pl
from jax.experimental.pallas import tpu as pltpu
from jax.experimental.pallas import tpu_sc as plsc

_N = 10000
_E = 320000
_D = 128
_DH = 64
_HW = 80
_CH = 80
_NW = 32
_EPW = _E // _NW
_EPT = _E // 16
_NCH = _EPT // _CH
_RPT = 640
_R = 2000


def _leaky(x):
    return jnp.where(x >= 0, x, 0.2 * x)


def _feature_tail(h, as_ref, ad_ref, h_ref, s_ref, d_ref):
    s_ref[...] = jnp.sum(h * as_ref[...], axis=1, keepdims=True)
    d_ref[...] = jnp.sum(h * ad_ref[...], axis=1, keepdims=True)
    ones = jnp.ones((_R, 1), jnp.float32)
    pad = jnp.zeros((_R, _HW - _DH - 1), jnp.float32)
    h_ref[0, ...] = jnp.concatenate([h[:, :_DH], ones, pad], axis=1)
    h_ref[1, ...] = jnp.concatenate([h[:, _DH:], ones, pad], axis=1)


def _prev_h(h_ref):
    return jnp.concatenate([h_ref[0, :, :_DH], h_ref[1, :, :_DH]], axis=1)


def _tc_pre_body(x_ref, w_ref, as_ref, ad_ref, h_ref, s_ref, d_ref):
    h = jnp.dot(x_ref[...], w_ref[...], preferred_element_type=jnp.float32)
    _feature_tail(h, as_ref, ad_ref, h_ref, s_ref, d_ref)


_ROWB = pl.BlockSpec((_R, 1), lambda i: (i, 0))
_H3B = pl.BlockSpec((2, _R, _HW), lambda i: (0, i, 0))


def _tc_pre(x, w, att_s, att_d):
    return pl.pallas_call(
        _tc_pre_body,
        grid=(_N // _R,),
        in_specs=[
            pl.BlockSpec((_R, _D), lambda i: (i, 0)),
            pl.BlockSpec((_D, _D), lambda i: (0, 0)),
            pl.BlockSpec((1, _D), lambda i: (0, 0)),
            pl.BlockSpec((1, _D), lambda i: (0, 0)),
        ],
        out_specs=(_H3B, _ROWB, _ROWB),
        out_shape=(
            jax.ShapeDtypeStruct((2, _N, _HW), jnp.float32),
            jax.ShapeDtypeStruct((_N, 1), jnp.float32),
            jax.ShapeDtypeStruct((_N, 1), jnp.float32),
        ),
    )(x, w, att_s, att_d)


def _normalized(acc_ref, h_ref, s_ref, d_ref, b_ref):
    den0 = jnp.exp(_leaky(s_ref[...] + d_ref[...]))
    num = jnp.concatenate([acc_ref[0, :, :_DH], acc_ref[1, :, :_DH]], axis=1)
    num = num + den0 * _prev_h(h_ref)
    den = acc_ref[0, :, _DH:_DH + 1] + den0
    return num / den + b_ref[...]


def _tc_mid_body(acc_ref, h_ref, s_ref, d_ref, b_ref, w_ref, as_ref, ad_ref,
                 h2_ref, s2_ref, d2_ref):
    h1 = jnp.maximum(_normalized(acc_ref, h_ref, s_ref, d_ref, b_ref), 0.0)
    h2 = jnp.dot(h1, w_ref[...], preferred_element_type=jnp.float32)
    _feature_tail(h2, as_ref, ad_ref, h2_ref, s2_ref, d2_ref)


def _tc_mid(acc, h, s, d, b, w, att_s, att_d):
    return pl.pallas_call(
        _tc_mid_body,
        grid=(_N // _R,),
        in_specs=[
            _H3B,
            _H3B,
            _ROWB,
            _ROWB,
            pl.BlockSpec((1, _D), lambda i: (0, 0)),
            pl.BlockSpec((_D, _D), lambda i: (0, 0)),
            pl.BlockSpec((1, _D), lambda i: (0, 0)),
            pl.BlockSpec((1, _D), lambda i: (0, 0)),
        ],
        out_specs=(_H3B, _ROWB, _ROWB),
        out_shape=(
            jax.ShapeDtypeStruct((2, _N, _HW), jnp.float32),
            jax.ShapeDtypeStruct((_N, 1), jnp.float32),
            jax.ShapeDtypeStruct((_N, 1), jnp.float32),
        ),
    )(acc, h, s, d, b, w, att_s, att_d)


def _tc_post_body(acc_ref, h_ref, s_ref, d_ref, b_ref, wp_ref, bp_ref,
                  ps_ref, pd_ref):
    hout = _normalized(acc_ref, h_ref, s_ref, d_ref, b_ref)
    ps_ref[...] = jnp.dot(hout, wp_ref[:_D, :],
                          preferred_element_type=jnp.float32) + bp_ref[...]
    pd_ref[...] = jnp.dot(hout, wp_ref[_D:, :],
                          preferred_element_type=jnp.float32)


def _tc_post(acc, h, s, d, b, wp, bp):
    return pl.pallas_call(
        _tc_post_body,
        grid=(_N // _R,),
        in_specs=[
            _H3B,
            _H3B,
            _ROWB,
            _ROWB,
            pl.BlockSpec((1, _D), lambda i: (0, 0)),
            pl.BlockSpec((2 * _D, 1), lambda i: (0, 0)),
            pl.BlockSpec((1, 1), lambda i: (0, 0)),
        ],
        out_specs=(_ROWB, _ROWB),
        out_shape=(
            jax.ShapeDtypeStruct((_N, 1), jnp.float32),
            jax.ShapeDtypeStruct((_N, 1), jnp.float32),
        ),
    )(acc, h, s, d, b, wp, bp)


_MESH = plsc.VectorSubcoreMesh(core_axis_name="c", subcore_axis_name="s")
_SC_PARAMS = pltpu.CompilerParams(needs_layout_passes=False,
                                  use_tc_tiling_on_sc=False)


@functools.partial(
    pl.kernel,
    out_type=jax.ShapeDtypeStruct((2, _N, _HW), jnp.float32),
    mesh=_MESH,
    compiler_params=_SC_PARAMS,
    scratch_types=[
        pltpu.VMEM((_N,), jnp.float32),
        pltpu.VMEM((_N,), jnp.float32),
        pltpu.VMEM((_EPT,), jnp.int32),
        pltpu.VMEM((_EPT,), jnp.int32),
        pltpu.VMEM((_CH,), jnp.int32),
        pltpu.VMEM((_CH,), jnp.int32),
        pltpu.VMEM((_CH,), jnp.float32),
        pltpu.VMEM((_CH,), jnp.float32),
        pltpu.VMEM((_CH, _HW), jnp.float32),
        pltpu.VMEM((_CH, _HW), jnp.float32),
        pltpu.VMEM_SHARED((_N, _HW), jnp.float32),
        pltpu.SemaphoreType.DMA,
        pltpu.SemaphoreType.DMA,
        pltpu.SemaphoreType.DMA,
        pltpu.SemaphoreType.DMA,
    ],
)
def _sc_agg(src_hbm, dst_hbm, s_hbm, d_hbm, h_hbm, z_hbm, acc_hbm,
            s_v, d_v, src_v, src2_v, dsta, dstb, exa, exb, rowsa, rowsb,
            num_sh, sema, semb, semia, semib):
    cid = lax.axis_index("c")
    sid = lax.axis_index("s")
    base = sid * _EPT

    pltpu.sync_copy(s_hbm, s_v)
    pltpu.sync_copy(d_hbm, d_v)
    pltpu.sync_copy(src_hbm.at[pl.ds(base, _EPT)], src_v)
    bias = jnp.broadcast_to(cid * _N, (16,)).astype(jnp.int32)

    @plsc.parallel_loop(0, _EPT // 16, unroll=8)
    def _(i):
        src2_v[pl.ds(i * 16, 16)] = src_v[pl.ds(i * 16, 16)] + bias

    @pl.when(sid < 15)
    def _():
        pltpu.sync_copy(z_hbm.at[pl.ds(sid * _RPT, _RPT)],
                        num_sh.at[pl.ds(sid * _RPT, _RPT)])

    @pl.when(sid == 15)
    def _():
        pltpu.sync_copy(z_hbm.at[pl.ds(15 * _RPT, _N - 15 * _RPT)],
                        num_sh.at[pl.ds(15 * _RPT, _N - 15 * _RPT)])

    plsc.subcore_barrier()

    def issue_gather(ci, rows, dstb_, sem, semi):
        idx = src2_v.at[pl.ds(ci * _CH, _CH)]
        pltpu.async_copy(h_hbm.at[idx], rows, sem)
        pltpu.async_copy(dst_hbm.at[pl.ds(base + ci * _CH, _CH)], dstb_, semi)

    def wait_gather(ci, rows, dstb_, sem, semi):
        idx = src2_v.at[pl.ds(ci * _CH, _CH)]
        pltpu.make_async_copy(h_hbm.at[idx], rows, sem).wait()
        pltpu.make_async_copy(
            dst_hbm.at[pl.ds(base + ci * _CH, _CH)], dstb_, semi).wait()

    def compute_ex(ci, ex, dstb_):
        for j in range(_CH // 16):
            si = src_v[pl.ds(ci * _CH + j * 16, 16)]
            di = dstb_[pl.ds(j * 16, 16)]
            a = plsc.load_gather(s_v, [si]) + plsc.load_gather(d_v, [di])
            ex[pl.ds(j * 16, 16)] = jnp.exp(_leaky(a))

    def scale_rows(rows, ex):
        @plsc.parallel_loop(0, _CH // 16, unroll=2)
        def _(g):
            er16 = ex[pl.ds(g * 16, 16)]
            for k in range(16):
                r = g * 16 + k
                er = jnp.full((16,), er16[k])
                for cb in range(_DH // 16):
                    rows[r, pl.ds(cb * 16, 16)] = (
                        rows[r, pl.ds(cb * 16, 16)] * er)
                rows[r, pl.ds(_DH, 16)] = er

    def scatter(rows, dstb_):
        pltpu.sync_copy(rows, num_sh.at[dstb_], add=True)

    def process(ci, ex, rows, dstb_):
        compute_ex(ci, ex, dstb_)
        scale_rows(rows, ex)
        scatter(rows, dstb_)

    issue_gather(0, rowsa, dsta, sema, semia)

    def pair(o, carry):
        e = o * 2
        wait_gather(e, rowsa, dsta, sema, semia)
        issue_gather(e + 1, rowsb, dstb, semb, semib)
        process(e, exa, rowsa, dsta)
        wait_gather(e + 1, rowsb, dstb, semb, semib)

        @pl.when(e + 2 < _NCH)
        def _():
            issue_gather(e + 2, rowsa, dsta, sema, semia)

        process(e + 1, exb, rowsb, dstb)
        return carry

    lax.fori_loop(0, _NCH // 2, pair, 0)

    plsc.subcore_barrier()

    @pl.when(sid < 15)
    def _():
        pltpu.sync_copy(num_sh.at[pl.ds(sid * _RPT, _RPT)],
                        acc_hbm.at[cid, pl.ds(sid * _RPT, _RPT)])

    @pl.when(sid == 15)
    def _():
        pltpu.sync_copy(num_sh.at[pl.ds(15 * _RPT, _N - 15 * _RPT)],
                        acc_hbm.at[cid, pl.ds(15 * _RPT, _N - 15 * _RPT)])


@functools.partial(
    pl.kernel,
    out_type=jax.ShapeDtypeStruct((_E,), jnp.float32),
    mesh=_MESH,
    compiler_params=_SC_PARAMS,
    scratch_types=[
        pltpu.VMEM((_N,), jnp.float32),
        pltpu.VMEM((_N,), jnp.float32),
        pltpu.VMEM((_EPW,), jnp.int32),
        pltpu.VMEM((_EPW,), jnp.int32),
        pltpu.VMEM((_EPW,), jnp.float32),
    ],
)
def _sc_score(src_hbm, dst_hbm, ps_hbm, pd_hbm, out_hbm,
              ps_v, pd_v, src_v, dst_v, out_v):
    cid = lax.axis_index("c")
    sid = lax.axis_index("s")
    wid = cid * 16 + sid

    pltpu.sync_copy(ps_hbm, ps_v)
    pltpu.sync_copy(pd_hbm, pd_v)
    pltpu.sync_copy(src_hbm.at[pl.ds(wid * _EPW, _EPW)], src_v)
    pltpu.sync_copy(dst_hbm.at[pl.ds(wid * _EPW, _EPW)], dst_v)

    def body(i, carry):
        si = src_v[pl.ds(i * 16, 16)]
        di = dst_v[pl.ds(i * 16, 16)]
        out_v[pl.ds(i * 16, 16)] = (plsc.load_gather(ps_v, [si])
                                    + plsc.load_gather(pd_v, [di]))
        return carry

    lax.fori_loop(0, _EPW // 16, body, 0)
    pltpu.sync_copy(out_v, out_hbm.at[pl.ds(wid * _EPW, _EPW)])


def kernel(x, edge_index, W1, att_s1, att_d1, b1, W2, att_s2, att_d2, b2,
           Wp, bp):
    src = edge_index[0]
    dst = edge_index[1]
    as1 = att_s1.reshape(1, _D)
    ad1 = att_d1.reshape(1, _D)
    as2 = att_s2.reshape(1, _D)
    ad2 = att_d2.reshape(1, _D)
    b1r = b1.reshape(1, _D)
    b2r = b2.reshape(1, _D)
    bpr = bp.reshape(1, 1)
    zeros = jnp.zeros((_N, _HW), jnp.float32)

    h1, s1, d1 = _tc_pre(x, W1, as1, ad1)
    acc1 = _sc_agg(src, dst, s1.reshape(_N), d1.reshape(_N),
                   h1.reshape(2 * _N, _HW), zeros)
    h2, s2, d2 = _tc_mid(acc1, h1, s1, d1, b1r, W2, as2, ad2)
    acc2 = _sc_agg(src, dst, s2.reshape(_N), d2.reshape(_N),
                   h2.reshape(2 * _N, _HW), zeros)
    ps, pd = _tc_post(acc2, h2, s2, d2, b2r, Wp, bpr)
    scores = _sc_score(src, dst, ps.reshape(_N), pd.reshape(_N))
    return scores.reshape(_E, 1)


# speedup vs baseline: 45.6123x; 1.2788x over previous
import functools

import jax
import jax.numpy as jnp
from jax import lax
from jax.experimental import pallas as pl
from jax.experimental.pallas import tpu as pltpu
from jax.experimental.pallas import tpu_sc as plsc

_N = 10000
_E = 320000
_D = 128
_DH = 64
_HW = 80
_CH = 80
_NW = 32
_EPW = _E // _NW
_EPT = _E // 16
_NCH = _EPT // _CH
_RPT = 640
_R = 2000


def _leaky(x):
    return jnp.where(x >= 0, x, 0.2 * x)


def _feature_tail(h, as_ref, ad_ref, h_ref, s_ref, d_ref):
    s_ref[...] = jnp.sum(h * as_ref[...], axis=1, keepdims=True)
    d_ref[...] = jnp.sum(h * ad_ref[...], axis=1, keepdims=True)
    ones = jnp.ones((_R, 1), jnp.float32)
    pad = jnp.zeros((_R, _HW - _DH - 1), jnp.float32)
    h_ref[0, ...] = jnp.concatenate([h[:, :_DH], ones, pad], axis=1)
    h_ref[1, ...] = jnp.concatenate([h[:, _DH:], ones, pad], axis=1)


def _prev_h(h_ref):
    return jnp.concatenate([h_ref[0, :, :_DH], h_ref[1, :, :_DH]], axis=1)


def _tc_pre_body(x_ref, w_ref, as_ref, ad_ref, h_ref, s_ref, d_ref):
    h = jnp.dot(x_ref[...], w_ref[...], preferred_element_type=jnp.float32)
    _feature_tail(h, as_ref, ad_ref, h_ref, s_ref, d_ref)


_ROWB = pl.BlockSpec((_R, 1), lambda i: (i, 0))
_H3B = pl.BlockSpec((2, _R, _HW), lambda i: (0, i, 0))


def _tc_pre(x, w, att_s, att_d):
    return pl.pallas_call(
        _tc_pre_body,
        grid=(_N // _R,),
        in_specs=[
            pl.BlockSpec((_R, _D), lambda i: (i, 0)),
            pl.BlockSpec((_D, _D), lambda i: (0, 0)),
            pl.BlockSpec((1, _D), lambda i: (0, 0)),
            pl.BlockSpec((1, _D), lambda i: (0, 0)),
        ],
        out_specs=(_H3B, _ROWB, _ROWB),
        out_shape=(
            jax.ShapeDtypeStruct((2, _N, _HW), jnp.float32),
            jax.ShapeDtypeStruct((_N, 1), jnp.float32),
            jax.ShapeDtypeStruct((_N, 1), jnp.float32),
        ),
    )(x, w, att_s, att_d)


def _normalized(acc_ref, h_ref, s_ref, d_ref, b_ref):
    den0 = jnp.exp(_leaky(s_ref[...] + d_ref[...]))
    num = jnp.concatenate([acc_ref[0, :, :_DH], acc_ref[1, :, :_DH]], axis=1)
    num = num + den0 * _prev_h(h_ref)
    den = acc_ref[0, :, _DH:_DH + 1] + den0
    return num / den + b_ref[...]


def _tc_mid_body(acc_ref, h_ref, s_ref, d_ref, b_ref, w_ref, as_ref, ad_ref,
                 h2_ref, s2_ref, d2_ref):
    h1 = jnp.maximum(_normalized(acc_ref, h_ref, s_ref, d_ref, b_ref), 0.0)
    h2 = jnp.dot(h1, w_ref[...], preferred_element_type=jnp.float32)
    _feature_tail(h2, as_ref, ad_ref, h2_ref, s2_ref, d2_ref)


def _tc_mid(acc, h, s, d, b, w, att_s, att_d):
    return pl.pallas_call(
        _tc_mid_body,
        grid=(_N // _R,),
        in_specs=[
            _H3B,
            _H3B,
            _ROWB,
            _ROWB,
            pl.BlockSpec((1, _D), lambda i: (0, 0)),
            pl.BlockSpec((_D, _D), lambda i: (0, 0)),
            pl.BlockSpec((1, _D), lambda i: (0, 0)),
            pl.BlockSpec((1, _D), lambda i: (0, 0)),
        ],
        out_specs=(_H3B, _ROWB, _ROWB),
        out_shape=(
            jax.ShapeDtypeStruct((2, _N, _HW), jnp.float32),
            jax.ShapeDtypeStruct((_N, 1), jnp.float32),
            jax.ShapeDtypeStruct((_N, 1), jnp.float32),
        ),
    )(acc, h, s, d, b, w, att_s, att_d)


def _tc_post_body(acc_ref, h_ref, s_ref, d_ref, b_ref, wp_ref, bp_ref,
                  ps_ref, pd_ref):
    hout = _normalized(acc_ref, h_ref, s_ref, d_ref, b_ref)
    ps_ref[...] = jnp.dot(hout, wp_ref[:_D, :],
                          preferred_element_type=jnp.float32) + bp_ref[...]
    pd_ref[...] = jnp.dot(hout, wp_ref[_D:, :],
                          preferred_element_type=jnp.float32)


def _tc_post(acc, h, s, d, b, wp, bp):
    return pl.pallas_call(
        _tc_post_body,
        grid=(_N // _R,),
        in_specs=[
            _H3B,
            _H3B,
            _ROWB,
            _ROWB,
            pl.BlockSpec((1, _D), lambda i: (0, 0)),
            pl.BlockSpec((2 * _D, 1), lambda i: (0, 0)),
            pl.BlockSpec((1, 1), lambda i: (0, 0)),
        ],
        out_specs=(_ROWB, _ROWB),
        out_shape=(
            jax.ShapeDtypeStruct((_N, 1), jnp.float32),
            jax.ShapeDtypeStruct((_N, 1), jnp.float32),
        ),
    )(acc, h, s, d, b, wp, bp)


_MESH = plsc.VectorSubcoreMesh(core_axis_name="c", subcore_axis_name="s")
_SC_PARAMS = pltpu.CompilerParams(needs_layout_passes=False,
                                  use_tc_tiling_on_sc=False)


@functools.partial(
    pl.kernel,
    out_type=jax.ShapeDtypeStruct((2, _N, _HW), jnp.float32),
    mesh=_MESH,
    compiler_params=_SC_PARAMS,
    scratch_types=[
        pltpu.VMEM((_EPT,), jnp.int32),
        pltpu.VMEM((_EPT,), jnp.int32),
        pltpu.VMEM((_EPT,), jnp.float32),
        pltpu.VMEM_SHARED((_N, _HW), jnp.float32),
        pltpu.SemaphoreType.DMA,
        pltpu.SemaphoreType.DMA,
        pltpu.SemaphoreType.DMA,
        pltpu.SemaphoreType.DMA,
        pltpu.SemaphoreType.DMA,
        pltpu.SemaphoreType.DMA,
    ],
)
def _sc_agg(src_hbm, dst_hbm, s_hbm, d_hbm, h_hbm, z_hbm, acc_hbm,
            src_v, dst_v, ex_v, num_sh, gs0, gs1, gs2, ss0, ss1, ss2):
    cid = lax.axis_index("c")
    sid = lax.axis_index("s")
    base = sid * _EPT
    gsem = [gs0, gs1, gs2]
    ssem = [ss0, ss1, ss2]

    pltpu.sync_copy(src_hbm.at[pl.ds(base, _EPT)], src_v)
    pltpu.sync_copy(dst_hbm.at[pl.ds(base, _EPT)], dst_v)

    def phase1(s_v, d_v):
        pltpu.sync_copy(s_hbm, s_v)
        pltpu.sync_copy(d_hbm, d_v)

        @plsc.parallel_loop(0, _EPT // 16, unroll=4)
        def _(i):
            si = src_v[pl.ds(i * 16, 16)]
            di = dst_v[pl.ds(i * 16, 16)]
            a = plsc.load_gather(s_v, [si]) + plsc.load_gather(d_v, [di])
            ex_v[pl.ds(i * 16, 16)] = jnp.exp(_leaky(a))

    pl.run_scoped(phase1, pltpu.VMEM((_N,), jnp.float32),
                  pltpu.VMEM((_N,), jnp.float32))

    bias = jnp.broadcast_to(cid * _N, (16,)).astype(jnp.int32)

    @plsc.parallel_loop(0, _EPT // 16, unroll=8)
    def _(i):
        src_v[pl.ds(i * 16, 16)] = src_v[pl.ds(i * 16, 16)] + bias

    @pl.when(sid < 15)
    def _():
        pltpu.sync_copy(z_hbm.at[pl.ds(sid * _RPT, _RPT)],
                        num_sh.at[pl.ds(sid * _RPT, _RPT)])

    @pl.when(sid == 15)
    def _():
        pltpu.sync_copy(z_hbm.at[pl.ds(15 * _RPT, _N - 15 * _RPT)],
                        num_sh.at[pl.ds(15 * _RPT, _N - 15 * _RPT)])

    plsc.subcore_barrier()

    def phase2(rows0, rows1, rows2):
        rowbuf = [rows0, rows1, rows2]

        def issue_gather(ci, q):
            idx = src_v.at[pl.ds(ci * _CH, _CH)]
            pltpu.async_copy(h_hbm.at[idx], rowbuf[q], gsem[q])

        def wait_gather(ci, q):
            idx = src_v.at[pl.ds(ci * _CH, _CH)]
            pltpu.make_async_copy(h_hbm.at[idx], rowbuf[q], gsem[q]).wait()

        def issue_scatter(ci, q):
            idx = dst_v.at[pl.ds(ci * _CH, _CH)]
            pltpu.async_copy(rowbuf[q], num_sh.at[idx], ssem[q], add=True)

        def wait_scatter(q):
            idx = dst_v.at[pl.ds(0, _CH)]
            pltpu.make_async_copy(rowbuf[q], num_sh.at[idx], ssem[q]).wait()

        def scale_rows(ci, q):
            rows = rowbuf[q]
            eb = ci * _CH

            @plsc.parallel_loop(0, _CH // 16, unroll=2)
            def _(g):
                er16 = ex_v[pl.ds(eb + g * 16, 16)]
                for k in range(16):
                    r = g * 16 + k
                    er = jnp.full((16,), er16[k])
                    for cb in range(_DH // 16):
                        rows[r, pl.ds(cb * 16, 16)] = (
                            rows[r, pl.ds(cb * 16, 16)] * er)
                    rows[r, pl.ds(_DH, 16)] = er

        issue_gather(0, 2)
        issue_gather(1, 0)
        wait_gather(0, 2)
        scale_rows(0, 2)
        issue_scatter(0, 2)
        issue_gather(2, 1)

        def triple(k, carry):
            for q in range(3):
                c = 1 + 3 * k + q
                pc = c + 2
                pq = (q + 2) % 3

                wait_gather(c, q)
                scale_rows(c, q)
                issue_scatter(c, q)

                @pl.when(pc < _NCH)
                def _():
                    wait_scatter(pq)
                    issue_gather(pc, pq)
            return carry

        lax.fori_loop(0, (_NCH - 1) // 3, triple, 0)
        for q in range(3):
            wait_scatter(q)

    pl.run_scoped(phase2,
                  pltpu.VMEM((_CH, _HW), jnp.float32),
                  pltpu.VMEM((_CH, _HW), jnp.float32),
                  pltpu.VMEM((_CH, _HW), jnp.float32))

    plsc.subcore_barrier()

    @pl.when(sid < 15)
    def _():
        pltpu.sync_copy(num_sh.at[pl.ds(sid * _RPT, _RPT)],
                        acc_hbm.at[cid, pl.ds(sid * _RPT, _RPT)])

    @pl.when(sid == 15)
    def _():
        pltpu.sync_copy(num_sh.at[pl.ds(15 * _RPT, _N - 15 * _RPT)],
                        acc_hbm.at[cid, pl.ds(15 * _RPT, _N - 15 * _RPT)])


@functools.partial(
    pl.kernel,
    out_type=jax.ShapeDtypeStruct((_E,), jnp.float32),
    mesh=_MESH,
    compiler_params=_SC_PARAMS,
    scratch_types=[
        pltpu.VMEM((_N,), jnp.float32),
        pltpu.VMEM((_N,), jnp.float32),
        pltpu.VMEM((_EPW,), jnp.int32),
        pltpu.VMEM((_EPW,), jnp.int32),
        pltpu.VMEM((_EPW,), jnp.float32),
    ],
)
def _sc_score(src_hbm, dst_hbm, ps_hbm, pd_hbm, out_hbm,
              ps_v, pd_v, src_v, dst_v, out_v):
    cid = lax.axis_index("c")
    sid = lax.axis_index("s")
    wid = cid * 16 + sid

    pltpu.sync_copy(ps_hbm, ps_v)
    pltpu.sync_copy(pd_hbm, pd_v)
    pltpu.sync_copy(src_hbm.at[pl.ds(wid * _EPW, _EPW)], src_v)
    pltpu.sync_copy(dst_hbm.at[pl.ds(wid * _EPW, _EPW)], dst_v)

    def body(i, carry):
        si = src_v[pl.ds(i * 16, 16)]
        di = dst_v[pl.ds(i * 16, 16)]
        out_v[pl.ds(i * 16, 16)] = (plsc.load_gather(ps_v, [si])
                                    + plsc.load_gather(pd_v, [di]))
        return carry

    lax.fori_loop(0, _EPW // 16, body, 0)
    pltpu.sync_copy(out_v, out_hbm.at[pl.ds(wid * _EPW, _EPW)])


def kernel(x, edge_index, W1, att_s1, att_d1, b1, W2, att_s2, att_d2, b2,
           Wp, bp):
    src = edge_index[0]
    dst = edge_index[1]
    as1 = att_s1.reshape(1, _D)
    ad1 = att_d1.reshape(1, _D)
    as2 = att_s2.reshape(1, _D)
    ad2 = att_d2.reshape(1, _D)
    b1r = b1.reshape(1, _D)
    b2r = b2.reshape(1, _D)
    bpr = bp.reshape(1, 1)
    zeros = jnp.zeros((_N, _HW), jnp.float32)

    h1, s1, d1 = _tc_pre(x, W1, as1, ad1)
    acc1 = _sc_agg(src, dst, s1.reshape(_N), d1.reshape(_N),
                   h1.reshape(2 * _N, _HW), zeros)
    h2, s2, d2 = _tc_mid(acc1, h1, s1, d1, b1r, W2, as2, ad2)
    acc2 = _sc_agg(src, dst, s2.reshape(_N), d2.reshape(_N),
                   h2.reshape(2 * _N, _HW), zeros)
    ps, pd = _tc_post(acc2, h2, s2, d2, b2r, Wp, bpr)
    scores = _sc_score(src, dst, ps.reshape(_N), pd.reshape(_N))
    return scores.reshape(_E, 1)
